# baseline (speedup 1.0000x reference)
import functools

import jax
import jax.numpy as jnp
from jax import lax
from jax.experimental import pallas as pl
from jax.experimental.pallas import tpu as pltpu
from jax.experimental.pallas import tpu_sc as plsc

B, S, T, C = 16, 64, 128, 256
K = 3
STEP = S // 3
NW = 32
HALF = T // 2
N_WIN_ITEMS = B * K * 2
N_SEG_ITEMS = B * K


def _body(win_hbm, seg_hbm, win_out, seg_out):
    wid = lax.axis_index("s") * 2 + lax.axis_index("c")

    for j in range(N_WIN_ITEMS // NW):
        i = wid + NW * j
        b = i // (K * 2)
        r = i % (K * 2)
        k = r // 2
        h = r % 2
        s = 1 + STEP * k
        pltpu.sync_copy(
            win_hbm.at[b, s, pl.ds(h * HALF, HALF)],
            win_out.at[b, k, pl.ds(h * HALF, HALF)],
        )

    for j in range(2):
        i = wid + NW * j

        @pl.when(i < N_SEG_ITEMS)
        def _():
            b = i // K
            k = i % K
            s = 1 + STEP * k
            pltpu.sync_copy(seg_hbm.at[b, s], seg_out.at[b, k])


@jax.jit
def _gather(win_feats, seg_feats):
    mesh = plsc.VectorSubcoreMesh(core_axis_name="c", subcore_axis_name="s")
    fn = functools.partial(
        pl.kernel,
        out_type=(
            jax.ShapeDtypeStruct((B, K, T, C), jnp.float32),
            jax.ShapeDtypeStruct((B, K, C), jnp.float32),
        ),
        mesh=mesh,
    )(_body)
    return fn(win_feats, seg_feats)


def kernel(feat, win_feats, seg_feats):
    del feat
    return _gather(win_feats, seg_feats)


# speedup vs baseline: 7.7878x; 7.7878x over previous
import functools

import jax
import jax.numpy as jnp
from jax import lax
from jax.experimental import pallas as pl
from jax.experimental.pallas import tpu as pltpu
from jax.experimental.pallas import tpu_sc as plsc

B, S, T, C = 16, 64, 128, 256
K = 3
STEP = S // 3
NW = 32
HALF = T // 2
N_WIN_ITEMS = B * K * 2
N_SEG_ITEMS = B * K
WPW = N_WIN_ITEMS // NW


def _win_idx(i):
    b = i // (K * 2)
    r = i % (K * 2)
    k = r // 2
    h = r % 2
    s = 1 + STEP * k
    return b, k, h, s


def _body(win_hbm, seg_hbm, win_out, seg_out,
          wbuf, sbuf, win_sems, wout_sems, seg_sems, sout_sems):
    wid = lax.axis_index("s") * 2 + lax.axis_index("c")

    win_in = []
    for j in range(WPW):
        b, k, h, s = _win_idx(wid + NW * j)
        win_in.append(pltpu.async_copy(
            win_hbm.at[b, s, pl.ds(h * HALF, HALF)], wbuf.at[j],
            win_sems.at[j]))
    i0 = wid
    b0, k0 = i0 // K, i0 % K
    seg_in0 = pltpu.async_copy(
        seg_hbm.at[b0, 1 + STEP * k0], sbuf.at[0], seg_sems.at[0])
    i1 = wid + NW

    @pl.when(i1 < N_SEG_ITEMS)
    def _():
        b1, k1 = i1 // K, i1 % K
        pltpu.async_copy(
            seg_hbm.at[b1, 1 + STEP * k1], sbuf.at[1], seg_sems.at[1])

    win_out_cps = []
    for j in range(WPW):
        win_in[j].wait()
        b, k, h, _ = _win_idx(wid + NW * j)
        win_out_cps.append(pltpu.async_copy(
            wbuf.at[j], win_out.at[b, k, pl.ds(h * HALF, HALF)],
            wout_sems.at[j]))

    seg_in0.wait()
    seg_out0 = pltpu.async_copy(
        sbuf.at[0], seg_out.at[b0, k0], sout_sems.at[0])

    @pl.when(i1 < N_SEG_ITEMS)
    def _():
        b1, k1 = i1 // K, i1 % K
        pltpu.make_async_copy(
            seg_hbm.at[b1, 1 + STEP * k1], sbuf.at[1], seg_sems.at[1]).wait()
        cp = pltpu.async_copy(
            sbuf.at[1], seg_out.at[b1, k1], sout_sems.at[1])
        cp.wait()

    for cp in win_out_cps:
        cp.wait()
    seg_out0.wait()


@jax.jit
def _gather(win_feats, seg_feats):
    mesh = plsc.VectorSubcoreMesh(core_axis_name="c", subcore_axis_name="s")
    fn = functools.partial(
        pl.kernel,
        out_type=(
            jax.ShapeDtypeStruct((B, K, T, C), jnp.float32),
            jax.ShapeDtypeStruct((B, K, C), jnp.float32),
        ),
        mesh=mesh,
        scratch_types=[
            pltpu.VMEM((WPW, HALF, C), jnp.float32),
            pltpu.VMEM((2, C), jnp.float32),
            pltpu.SemaphoreType.DMA((WPW,)),
            pltpu.SemaphoreType.DMA((WPW,)),
            pltpu.SemaphoreType.DMA((2,)),
            pltpu.SemaphoreType.DMA((2,)),
        ],
    )(_body)
    return fn(win_feats, seg_feats)


def kernel(feat, win_feats, seg_feats):
    del feat
    return _gather(win_feats, seg_feats)
